# baseline (speedup 1.0000x reference)
import functools

import jax
import jax.numpy as jnp
from jax import lax
from jax.experimental import pallas as pl
from jax.experimental.pallas import tpu as pltpu
from jax.experimental.pallas import tpu_sc as plsc

N = 10000
E = 320000
D = 128
H = 128
G = 64
L = 3
NP = 10240
NB = 2048
NBLK = NP // NB
EP = 321536
EB = 2048
EBLK = EP // EB
C = 2048
NCH = EP // C
F32 = jnp.float32

_dg = functools.partial(lax.dot_general, preferred_element_type=F32)
_dgh = functools.partial(lax.dot_general, preferred_element_type=F32,
                         precision=lax.Precision.HIGHEST)


def _dgc(a, b, ca, cb):
    return _dg(a, b, (((ca,), (cb,)), ((), ())))


def _dgch(a, b, ca, cb):
    return _dgh(a, b, (((ca,), (cb,)), ((), ())))


def _k1_body(x_ref, projW, projBc, nodeW, nodeBc, msgWx, hT_ref, aT_ref):
    xb = x_ref[...]
    hT = _dgc(projW[...], xb, 0, 1) + projBc[...]
    hnT = _dgc(nodeW[...], hT, 0, 0) + nodeBc[...]
    aT_ref[...] = _dgc(msgWx[...], hnT, 0, 0)
    hT_ref[...] = hT


def _k1(x_pad, projW, projBc, nodeW, nodeBc, msgWx):
    full = lambda shp: pl.BlockSpec(shp, lambda i: tuple(0 for _ in shp))
    return pl.pallas_call(
        _k1_body,
        grid=(NBLK,),
        in_specs=[pl.BlockSpec((NB, D), lambda i: (i, 0)),
                  full((D, H)), full((H, 1)), full((H, H)), full((H, 1)),
                  full((H, H))],
        out_specs=[pl.BlockSpec((H, NB), lambda i: (0, i)),
                   pl.BlockSpec((H, NB), lambda i: (0, i))],
        out_shape=[jax.ShapeDtypeStruct((H, NP), F32),
                   jax.ShapeDtypeStruct((H, NP), F32)],
    )(x_pad, projW, projBc, nodeW, nodeBc, msgWx)


def _ket_body(ea_ref, ewS, ebS, weS, mbS, etT_ref):
    ef = _dg(ea_ref[...], ewS[0], (((1,), (0,)), ((), ()))) + ebS[0]
    etT_ref[0] = _dgc(weS[0], ef, 0, 1) + mbS[0]


def _ket(ea_pad, ewS, ebS, weS, mbS):
    return pl.pallas_call(
        _ket_body,
        grid=(L, EBLK),
        in_specs=[pl.BlockSpec((EB, 4), lambda l, j: (j, 0)),
                  pl.BlockSpec((1, 4, H), lambda l, j: (l, 0, 0)),
                  pl.BlockSpec((1, 1, H), lambda l, j: (l, 0, 0)),
                  pl.BlockSpec((1, H, H), lambda l, j: (l, 0, 0)),
                  pl.BlockSpec((1, H, 1), lambda l, j: (l, 0, 0))],
        out_specs=pl.BlockSpec((1, H, EB), lambda l, j: (l, 0, j)),
        out_shape=jax.ShapeDtypeStruct((L, H, EP), F32),
    )(ea_pad, ewS, ebS, weS, mbS)


def _sc_edge_body(a_hbm, src_hbm, dst_hbm, et_hbm, agg_hbm,
                  a_v, agg_v, src_v, dst_v, et_v):
    c_i = lax.axis_index("c")
    s_i = lax.axis_index("s")
    w = s_i * 2 + c_i
    fb = w * 4
    for k in range(4):
        pltpu.sync_copy(a_hbm.at[pl.ds((fb + k) * NP, NP)], a_v.at[k])

    def zb(i, _):
        z = jnp.zeros((16,), F32)
        for k in range(4):
            agg_v[k, pl.ds(i * 16, 16)] = z
        return 0
    lax.fori_loop(0, NP // 16, zb, 0)

    def chunk(ci, _):
        cc = lax.rem(ci + w * 5, NCH)
        e0 = cc * C
        pltpu.sync_copy(src_hbm.at[pl.ds(e0, C)], src_v)
        pltpu.sync_copy(dst_hbm.at[pl.ds(e0, C)], dst_v)
        for k in range(4):
            pltpu.sync_copy(et_hbm.at[pl.ds((fb + k) * EP + e0, C)],
                            et_v.at[k])

        def blk(j, _):
            b0 = j * 16
            srcv = src_v[pl.ds(b0, 16)]
            dstv = dst_v[pl.ds(b0, 16)]
            for k in range(4):
                kv = jnp.full((16,), k, jnp.int32)
                et = et_v[k, pl.ds(b0, 16)]
                av = plsc.load_gather(a_v, [kv, srcv])
                m = jnp.maximum(av + et, 0.0)
                plsc.addupdate_scatter(agg_v, [kv, dstv], m)
            return 0
        lax.fori_loop(0, C // 16, blk, 0)
        return 0
    lax.fori_loop(0, NCH, chunk, 0)
    for k in range(4):
        pltpu.sync_copy(agg_v.at[k], agg_hbm.at[pl.ds((fb + k) * NP, NP)])


@functools.cache
def _sc_edge_kernel():
    return pl.kernel(
        _sc_edge_body,
        out_type=jax.ShapeDtypeStruct((H * NP,), F32),
        mesh=plsc.VectorSubcoreMesh(core_axis_name="c", subcore_axis_name="s"),
        scratch_types=[pltpu.VMEM((4, NP), F32),
                       pltpu.VMEM((4, NP), F32),
                       pltpu.VMEM((C,), jnp.int32),
                       pltpu.VMEM((C,), jnp.int32),
                       pltpu.VMEM((4, C), F32)],
        compiler_params=pltpu.CompilerParams(needs_layout_passes=False),
    )


def _k2_body(hT_ref, aggT_ref, lnG, lnB, nodeW, nodeBc, msgWx,
             hTn_ref, aT_ref):
    u = hT_ref[...] + aggT_ref[...]
    mu = jnp.mean(u, axis=0, keepdims=True)
    var = jnp.mean(jnp.square(u - mu), axis=0, keepdims=True)
    hTn = lnG[...] * (u - mu) / jnp.sqrt(var + 1e-5) + lnB[...]
    hTn_ref[...] = hTn
    hnT = _dgc(nodeW[...], hTn, 0, 0) + nodeBc[...]
    aT_ref[...] = _dgc(msgWx[...], hnT, 0, 0)


def _k2(hT, aggT, lnG, lnB, nodeW, nodeBc, msgWx):
    full = lambda shp: pl.BlockSpec(shp, lambda i: tuple(0 for _ in shp))
    blk = pl.BlockSpec((H, NB), lambda i: (0, i))
    return pl.pallas_call(
        _k2_body,
        grid=(NBLK,),
        in_specs=[blk, blk, full((H, 1)), full((H, 1)), full((H, H)),
                  full((H, 1)), full((H, H))],
        out_specs=[blk, blk],
        out_shape=[jax.ShapeDtypeStruct((H, NP), F32),
                   jax.ShapeDtypeStruct((H, NP), F32)],
    )(hT, aggT, lnG, lnB, nodeW, nodeBc, msgWx)


def _k3_body(hT_ref, aggT_ref, lnG, lnB, batch_ref,
             pfW1, pfB1, pfW2, pfB2, genW1, genB1, genW2, genB2,
             attW1, attB1, attW2, attB2,
             h_ref, vmag_ref, s_ref, c_ref, gen_ref, att_ref,
             m_ref, ssum_ref, cnt_ref):
    i = pl.program_id(0)
    u = hT_ref[...] + aggT_ref[...]
    mu = jnp.mean(u, axis=0, keepdims=True)
    var = jnp.mean(jnp.square(u - mu), axis=0, keepdims=True)
    hTn = lnG[...] * (u - mu) / jnp.sqrt(var + 1e-5) + lnB[...]
    eye = (lax.broadcasted_iota(jnp.int32, (H, H), 0)
           == lax.broadcasted_iota(jnp.int32, (H, H), 1)).astype(F32)
    hb = _dgch(hTn, eye, 0, 0)
    h_ref[...] = hb
    pf = _dg(jax.nn.relu(_dg(hb, pfW1[...], (((1,), (0,)), ((), ())))
                         + pfB1[...]),
             pfW2[...], (((1,), (0,)), ((), ()))) + pfB2[...]
    pf0 = pf[:, 0:1]
    vmag_ref[...] = jnp.maximum(pf0, 0.0) + jnp.log1p(jnp.exp(-jnp.abs(pf0)))
    s0 = pf[:, 1:2]
    c0 = pf[:, 2:3]
    nrm = jnp.sqrt(s0 * s0 + c0 * c0 + 1e-8)
    s_ref[...] = s0 / nrm
    c_ref[...] = c0 / nrm
    gen_ref[...] = _dg(jax.nn.relu(_dg(hb, genW1[...], (((1,), (0,)), ((), ())))
                                   + genB1[...]),
                       genW2[...], (((1,), (0,)), ((), ()))) + genB2[...]
    attT = _dgc(attW2[...], jnp.tanh(_dgc(attW1[...], hTn, 0, 0) + attB1[...]),
                0, 0) + attB2[...]
    att_ref[...] = attT
    bb = batch_ref[...]
    gio = lax.broadcasted_iota(jnp.int32, (G, NB), 0).astype(F32)
    mask = (gio == bb).astype(F32)
    bm = jnp.max(jnp.where(mask > 0.0, attT, -1e30), axis=1, keepdims=True)
    bs = _dgch(mask, hb, 1, 0)
    bc = jnp.sum(mask, axis=1, keepdims=True)

    @pl.when(i == 0)
    def _():
        m_ref[...] = bm
        ssum_ref[...] = bs
        cnt_ref[...] = bc

    @pl.when(i > 0)
    def _():
        m_ref[...] = jnp.maximum(m_ref[...], bm)
        ssum_ref[...] += bs
        cnt_ref[...] += bc


def _k3(hT, aggT, lnG, lnB, batch_row, hw):
    full = lambda shp: pl.BlockSpec(shp, lambda i: tuple(0 for _ in shp))
    blkT = pl.BlockSpec((H, NB), lambda i: (0, i))
    blkR = pl.BlockSpec((1, NB), lambda i: (0, i))
    return pl.pallas_call(
        _k3_body,
        grid=(NBLK,),
        in_specs=[blkT, blkT, full((H, 1)), full((H, 1)), blkR]
                 + [full(a.shape) for a in hw],
        out_specs=[pl.BlockSpec((NB, H), lambda i: (i, 0)),
                   pl.BlockSpec((NB, 1), lambda i: (i, 0)),
                   pl.BlockSpec((NB, 1), lambda i: (i, 0)),
                   pl.BlockSpec((NB, 1), lambda i: (i, 0)),
                   pl.BlockSpec((NB, 2), lambda i: (i, 0)),
                   blkR,
                   full((G, 1)), full((G, H)), full((G, 1))],
        out_shape=[jax.ShapeDtypeStruct((NP, H), F32),
                   jax.ShapeDtypeStruct((NP, 1), F32),
                   jax.ShapeDtypeStruct((NP, 1), F32),
                   jax.ShapeDtypeStruct((NP, 1), F32),
                   jax.ShapeDtypeStruct((NP, 2), F32),
                   jax.ShapeDtypeStruct((1, NP), F32),
                   jax.ShapeDtypeStruct((G, 1), F32),
                   jax.ShapeDtypeStruct((G, H), F32),
                   jax.ShapeDtypeStruct((G, 1), F32)],
    )(hT, aggT, lnG, lnB, batch_row, *hw)


def _k5_body(att_ref, batch_ref, m_ref, ex_ref, den_ref):
    i = pl.program_id(0)
    att = att_ref[...]
    bb = batch_ref[...]
    gio = lax.broadcasted_iota(jnp.int32, (G, NB), 0).astype(F32)
    mask = (gio == bb).astype(F32)
    mcol = _dgch(m_ref[...], mask, 0, 0)
    valid = bb < float(G)
    ex = jnp.where(valid, jnp.exp(att - mcol), 0.0)
    ex_ref[...] = ex
    bden = _dgch(mask, ex, 1, 1)

    @pl.when(i == 0)
    def _():
        den_ref[...] = bden

    @pl.when(i > 0)
    def _():
        den_ref[...] += bden


def _k5(att_row, batch_row, m):
    full = lambda shp: pl.BlockSpec(shp, lambda i: tuple(0 for _ in shp))
    blkR = pl.BlockSpec((1, NB), lambda i: (0, i))
    return pl.pallas_call(
        _k5_body,
        grid=(NBLK,),
        in_specs=[blkR, blkR, full((G, 1))],
        out_specs=[blkR, full((G, 1))],
        out_shape=[jax.ShapeDtypeStruct((1, NP), F32),
                   jax.ShapeDtypeStruct((G, 1), F32)],
    )(att_row, batch_row, m)


def _k6_body(ex_ref, batch_ref, den_ref, h_ref, ssum_ref, cnt_ref,
             costW1, costB1, costW2, costB2, clsW1, clsB1, clsW2, clsB2,
             attw_ref, gsum_ref, cost_ref, logits_ref):
    i = pl.program_id(0)
    bb = batch_ref[...]
    gio = lax.broadcasted_iota(jnp.int32, (G, NB), 0).astype(F32)
    mask = (gio == bb).astype(F32)
    denc = _dgch(den_ref[...], mask, 0, 0)
    valid = bb < float(G)
    attw = jnp.where(valid, ex_ref[...] / jnp.maximum(denc, 1e-30), 0.0)
    attw_ref[...] = attw
    bg = _dgch(mask * attw, h_ref[...], 1, 0)

    @pl.when(i == 0)
    def _():
        gsum_ref[...] = bg

    @pl.when(i > 0)
    def _():
        gsum_ref[...] += bg

    gs = gsum_ref[...]
    gmean = ssum_ref[...] / jnp.maximum(cnt_ref[...], 1.0)
    cost_ref[...] = _dg(jax.nn.relu(
        _dg(gmean, costW1[...], (((1,), (0,)), ((), ()))) + costB1[...]),
        costW2[...], (((1,), (0,)), ((), ()))) + costB2[...]
    logits_ref[...] = _dg(jax.nn.relu(
        _dg(gs, clsW1[...], (((1,), (0,)), ((), ()))) + clsB1[...]),
        clsW2[...], (((1,), (0,)), ((), ()))) + clsB2[...]


def _k6(ex_row, batch_row, den, h, ssum, cnt, hw):
    full = lambda shp: pl.BlockSpec(shp, lambda i: tuple(0 for _ in shp))
    blkR = pl.BlockSpec((1, NB), lambda i: (0, i))
    return pl.pallas_call(
        _k6_body,
        grid=(NBLK,),
        in_specs=[blkR, blkR, full((G, 1)),
                  pl.BlockSpec((NB, H), lambda i: (i, 0)),
                  full((G, H)), full((G, 1))] + [full(a.shape) for a in hw],
        out_specs=[blkR, full((G, H)), full((G, 1)), full((G, 3))],
        out_shape=[jax.ShapeDtypeStruct((1, NP), F32),
                   jax.ShapeDtypeStruct((G, H), F32),
                   jax.ShapeDtypeStruct((G, 1), F32),
                   jax.ShapeDtypeStruct((G, 3), F32)],
    )(ex_row, batch_row, den, h, ssum, cnt, *hw)


def _edge_stage(a_flat, src, dst, et_flat):
    return _sc_edge_kernel()(a_flat, src, dst, et_flat)


def kernel(x, edge_index, edge_attr, batch, params):
    p = params
    x_pad = jnp.pad(x, ((0, NP - N), (0, 0)))
    batch_row = jnp.pad(batch, (0, NP - N), constant_values=G) \
                   .astype(F32).reshape(1, NP)
    src = jnp.pad(edge_index[0].astype(jnp.int32), (0, EP - E))
    dst = jnp.pad(edge_index[1].astype(jnp.int32), (0, EP - E),
                  constant_values=NP - 1)

    lps = p['layers']
    ea_pad = jnp.pad(edge_attr, ((0, EP - E), (0, 0)))
    ewS = jnp.stack([lp['edge_W'] for lp in lps])
    ebS = jnp.stack([lp['edge_b'].reshape(1, H) for lp in lps])
    weS = jnp.stack([lp['msg_W'][H:] for lp in lps])
    mbS = jnp.stack([lp['msg_b'].reshape(H, 1) for lp in lps])
    etT = _ket(ea_pad, ewS, ebS, weS, mbS)

    hT, aT = _k1(x_pad, p['proj_W'], p['proj_b'].reshape(H, 1),
                 lps[0]['node_W'], lps[0]['node_b'].reshape(H, 1),
                 lps[0]['msg_W'][:H])

    for l in range(L):
        aggT = _edge_stage(aT.reshape(-1), src, dst,
                           etT[l].reshape(-1)).reshape(H, NP)
        if l < L - 1:
            lp = lps[l + 1]
            hT, aT = _k2(hT, aggT, lps[l]['ln_g'].reshape(H, 1),
                         lps[l]['ln_b'].reshape(H, 1),
                         lp['node_W'], lp['node_b'].reshape(H, 1),
                         lp['msg_W'][:H])

    hw3 = [p['pf_W1'], p['pf_b1'].reshape(1, H), p['pf_W2'],
           p['pf_b2'].reshape(1, 3),
           p['gen_W1'], p['gen_b1'].reshape(1, H), p['gen_W2'],
           p['gen_b2'].reshape(1, 2),
           p['att_W1'], p['att_b1'].reshape(H // 2, 1), p['att_W2'],
           p['att_b2'].reshape(1, 1)]
    (h, vmag, s, c, gen, att_row, m, ssum, cnt) = _k3(
        hT, aggT, lps[2]['ln_g'].reshape(H, 1), lps[2]['ln_b'].reshape(H, 1),
        batch_row, hw3)

    ex_row, den = _k5(att_row, batch_row, m)

    hw6 = [p['cost_W1'], p['cost_b1'].reshape(1, H // 2), p['cost_W2'],
           p['cost_b2'].reshape(1, 1),
           p['cls_W1'], p['cls_b1'].reshape(1, H // 2), p['cls_W2'],
           p['cls_b2'].reshape(1, 3)]
    attw_row, _gsum, cost, logits = _k6(ex_row, batch_row, den, h, ssum,
                                        cnt, hw6)

    return (vmag[:N], s[:N], c[:N], gen[:N], cost, logits,
            attw_row.reshape(NP, 1)[:N], h[:N])


# speedup vs baseline: 1.3537x; 1.3537x over previous
import functools

import jax
import jax.numpy as jnp
from jax import lax
from jax.experimental import pallas as pl
from jax.experimental.pallas import tpu as pltpu
from jax.experimental.pallas import tpu_sc as plsc

N = 10000
E = 320000
D = 128
H = 128
G = 64
L = 3
NP = 10240
NB = 2048
NBLK = NP // NB
EP = 321536
C = 1024
NCH = EP // C
EB = C
EBLK = NCH
F32 = jnp.float32

_dg = functools.partial(lax.dot_general, preferred_element_type=F32)
_dgh = functools.partial(lax.dot_general, preferred_element_type=F32,
                         precision=lax.Precision.HIGHEST)


def _dgc(a, b, ca, cb):
    return _dg(a, b, (((ca,), (cb,)), ((), ())))


def _dgch(a, b, ca, cb):
    return _dgh(a, b, (((ca,), (cb,)), ((), ())))


def _k1_body(x_ref, projW, projBc, nodeW, nodeBc, msgWx, hT_ref, aT_ref):
    xb = x_ref[...]
    hT = _dgc(projW[...], xb, 0, 1) + projBc[...]
    hnT = _dgc(nodeW[...], hT, 0, 0) + nodeBc[...]
    aT_ref[...] = _dgc(msgWx[...], hnT, 0, 0)
    hT_ref[...] = hT


def _k1(x_pad, projW, projBc, nodeW, nodeBc, msgWx):
    full = lambda shp: pl.BlockSpec(shp, lambda i: tuple(0 for _ in shp))
    return pl.pallas_call(
        _k1_body,
        grid=(NBLK,),
        in_specs=[pl.BlockSpec((NB, D), lambda i: (i, 0)),
                  full((D, H)), full((H, 1)), full((H, H)), full((H, 1)),
                  full((H, H))],
        out_specs=[pl.BlockSpec((H, NB), lambda i: (0, i)),
                   pl.BlockSpec((H, NB), lambda i: (0, i))],
        out_shape=[jax.ShapeDtypeStruct((H, NP), F32),
                   jax.ShapeDtypeStruct((H, NP), F32)],
    )(x_pad, projW, projBc, nodeW, nodeBc, msgWx)


def _ket_body(ea_ref, ewS, ebS, weS, mbS, etT_ref):
    ef = _dg(ea_ref[...], ewS[0], (((1,), (0,)), ((), ()))) + ebS[0]
    etT_ref[0, 0] = _dgc(weS[0], ef, 0, 1) + mbS[0]


def _ket(ea_pad, ewS, ebS, weS, mbS):
    return pl.pallas_call(
        _ket_body,
        grid=(L, EBLK),
        in_specs=[pl.BlockSpec((EB, 4), lambda l, j: (j, 0)),
                  pl.BlockSpec((1, 4, H), lambda l, j: (l, 0, 0)),
                  pl.BlockSpec((1, 1, H), lambda l, j: (l, 0, 0)),
                  pl.BlockSpec((1, H, H), lambda l, j: (l, 0, 0)),
                  pl.BlockSpec((1, H, 1), lambda l, j: (l, 0, 0))],
        out_specs=pl.BlockSpec((1, 1, H, EB), lambda l, j: (l, j, 0, 0)),
        out_shape=jax.ShapeDtypeStruct((L, EBLK, H, EB), F32),
    )(ea_pad, ewS, ebS, weS, mbS)


def _kpk_body(src_ref, dst_ref, pk_ref):
    pk_ref[...] = src_ref[...] + dst_ref[...] * 16384


def _kpk(src2, dst2):
    full = lambda shp: pl.BlockSpec(shp, lambda: tuple(0 for _ in shp))
    return pl.pallas_call(
        _kpk_body,
        in_specs=[full(src2.shape), full(src2.shape)],
        out_specs=full(src2.shape),
        out_shape=jax.ShapeDtypeStruct(src2.shape, jnp.int32),
    )(src2, dst2)


def _sc_edge_body(a_hbm, pk_hbm, et_hbm, agg_hbm,
                  a_v, agg_v, pk_v0, pk_v1, et_v0, et_v1, sem0, sem1):
    c_i = lax.axis_index("c")
    s_i = lax.axis_index("s")
    w = s_i * 2 + c_i
    fb = w * 4
    pltpu.sync_copy(a_hbm.at[pl.ds(fb * NP, 4 * NP)], a_v)

    def zb(i, _):
        agg_v[pl.ds(i * 16, 16)] = jnp.zeros((16,), F32)
        return 0
    lax.fori_loop(0, (4 * NP) // 16, zb, 0)

    bufs = ((pk_v0, et_v0, sem0), (pk_v1, et_v1, sem1))

    def start(g, b):
        cc = g
        pk_v, et_v, sem = bufs[b]
        pltpu.make_async_copy(pk_hbm.at[pl.ds(cc * C, C)], pk_v, sem).start()
        pltpu.make_async_copy(et_hbm.at[pl.ds(cc * (H * C) + fb * C, 4 * C)],
                              et_v, sem).start()

    def wait(b):
        pk_v, et_v, sem = bufs[b]
        pltpu.make_async_copy(pk_hbm.at[pl.ds(0, C)], pk_v, sem).wait()
        pltpu.make_async_copy(et_hbm.at[pl.ds(0, 4 * C)], et_v, sem).wait()

    def compute(b):
        pk_v, et_v, _ = bufs[b]

        def blk(j, _):
            b0 = j * 16
            p = pk_v[pl.ds(b0, 16)]
            srcv = lax.bitwise_and(p, 16383)
            dstv = lax.shift_right_logical(p, 14)
            for k in range(4):
                et = et_v[pl.ds(k * C + b0, 16)]
                av = plsc.load_gather(a_v, [srcv + (k * NP) if k else srcv])
                m = jnp.maximum(av + et, 0.0)
                plsc.addupdate_scatter(agg_v,
                                       [dstv + (k * NP) if k else dstv], m)
            return 0
        lax.fori_loop(0, C // 16, blk, 0)

    start(0, 0)
    start(1, 1)

    def pair(g2, _):
        g0 = g2 * 2
        for b in range(2):
            wait(b)
            compute(b)

            @pl.when(g0 + b + 2 < NCH)
            def _():
                start(g0 + b + 2, b)
        return 0
    lax.fori_loop(0, NCH // 2, pair, 0)
    pltpu.sync_copy(agg_v, agg_hbm.at[pl.ds(fb * NP, 4 * NP)])


@functools.cache
def _sc_edge_kernel():
    return pl.kernel(
        _sc_edge_body,
        out_type=jax.ShapeDtypeStruct((H * NP,), F32),
        mesh=plsc.VectorSubcoreMesh(core_axis_name="c", subcore_axis_name="s"),
        scratch_types=[pltpu.VMEM((4 * NP,), F32),
                       pltpu.VMEM((4 * NP,), F32),
                       pltpu.VMEM((C,), jnp.int32),
                       pltpu.VMEM((C,), jnp.int32),
                       pltpu.VMEM((4 * C,), F32),
                       pltpu.VMEM((4 * C,), F32),
                       pltpu.SemaphoreType.DMA,
                       pltpu.SemaphoreType.DMA],
        compiler_params=pltpu.CompilerParams(needs_layout_passes=False),
    )


def _k2_body(hT_ref, aggT_ref, lnG, lnB, nodeW, nodeBc, msgWx,
             hTn_ref, aT_ref):
    u = hT_ref[...] + aggT_ref[...]
    mu = jnp.mean(u, axis=0, keepdims=True)
    var = jnp.mean(jnp.square(u - mu), axis=0, keepdims=True)
    hTn = lnG[...] * (u - mu) / jnp.sqrt(var + 1e-5) + lnB[...]
    hTn_ref[...] = hTn
    hnT = _dgc(nodeW[...], hTn, 0, 0) + nodeBc[...]
    aT_ref[...] = _dgc(msgWx[...], hnT, 0, 0)


def _k2(hT, aggT, lnG, lnB, nodeW, nodeBc, msgWx):
    full = lambda shp: pl.BlockSpec(shp, lambda i: tuple(0 for _ in shp))
    blk = pl.BlockSpec((H, NB), lambda i: (0, i))
    return pl.pallas_call(
        _k2_body,
        grid=(NBLK,),
        in_specs=[blk, blk, full((H, 1)), full((H, 1)), full((H, H)),
                  full((H, 1)), full((H, H))],
        out_specs=[blk, blk],
        out_shape=[jax.ShapeDtypeStruct((H, NP), F32),
                   jax.ShapeDtypeStruct((H, NP), F32)],
    )(hT, aggT, lnG, lnB, nodeW, nodeBc, msgWx)


def _k3_body(hT_ref, aggT_ref, lnG, lnB, batch_ref,
             pfW1, pfB1, pfW2, pfB2, genW1, genB1, genW2, genB2,
             attW1, attB1, attW2, attB2,
             h_ref, vmag_ref, s_ref, c_ref, gen_ref, att_ref,
             m_ref, ssum_ref, cnt_ref):
    i = pl.program_id(0)
    u = hT_ref[...] + aggT_ref[...]
    mu = jnp.mean(u, axis=0, keepdims=True)
    var = jnp.mean(jnp.square(u - mu), axis=0, keepdims=True)
    hTn = lnG[...] * (u - mu) / jnp.sqrt(var + 1e-5) + lnB[...]
    eye = (lax.broadcasted_iota(jnp.int32, (H, H), 0)
           == lax.broadcasted_iota(jnp.int32, (H, H), 1)).astype(F32)
    hb = _dgch(hTn, eye, 0, 0)
    h_ref[...] = hb
    pf = _dg(jax.nn.relu(_dg(hb, pfW1[...], (((1,), (0,)), ((), ())))
                         + pfB1[...]),
             pfW2[...], (((1,), (0,)), ((), ()))) + pfB2[...]
    pf0 = pf[:, 0:1]
    vmag_ref[...] = jnp.maximum(pf0, 0.0) + jnp.log1p(jnp.exp(-jnp.abs(pf0)))
    s0 = pf[:, 1:2]
    c0 = pf[:, 2:3]
    nrm = jnp.sqrt(s0 * s0 + c0 * c0 + 1e-8)
    s_ref[...] = s0 / nrm
    c_ref[...] = c0 / nrm
    gen_ref[...] = _dg(jax.nn.relu(_dg(hb, genW1[...], (((1,), (0,)), ((), ())))
                                   + genB1[...]),
                       genW2[...], (((1,), (0,)), ((), ()))) + genB2[...]
    attT = _dgc(attW2[...], jnp.tanh(_dgc(attW1[...], hTn, 0, 0) + attB1[...]),
                0, 0) + attB2[...]
    att_ref[...] = attT
    bb = batch_ref[...]
    gio = lax.broadcasted_iota(jnp.int32, (G, NB), 0).astype(F32)
    mask = (gio == bb).astype(F32)
    bm = jnp.max(jnp.where(mask > 0.0, attT, -1e30), axis=1, keepdims=True)
    bs = _dgch(mask, hb, 1, 0)
    bc = jnp.sum(mask, axis=1, keepdims=True)

    @pl.when(i == 0)
    def _():
        m_ref[...] = bm
        ssum_ref[...] = bs
        cnt_ref[...] = bc

    @pl.when(i > 0)
    def _():
        m_ref[...] = jnp.maximum(m_ref[...], bm)
        ssum_ref[...] += bs
        cnt_ref[...] += bc


def _k3(hT, aggT, lnG, lnB, batch_row, hw):
    full = lambda shp: pl.BlockSpec(shp, lambda i: tuple(0 for _ in shp))
    blkT = pl.BlockSpec((H, NB), lambda i: (0, i))
    blkR = pl.BlockSpec((1, NB), lambda i: (0, i))
    return pl.pallas_call(
        _k3_body,
        grid=(NBLK,),
        in_specs=[blkT, blkT, full((H, 1)), full((H, 1)), blkR]
                 + [full(a.shape) for a in hw],
        out_specs=[pl.BlockSpec((NB, H), lambda i: (i, 0)),
                   pl.BlockSpec((NB, 1), lambda i: (i, 0)),
                   pl.BlockSpec((NB, 1), lambda i: (i, 0)),
                   pl.BlockSpec((NB, 1), lambda i: (i, 0)),
                   pl.BlockSpec((NB, 2), lambda i: (i, 0)),
                   blkR,
                   full((G, 1)), full((G, H)), full((G, 1))],
        out_shape=[jax.ShapeDtypeStruct((NP, H), F32),
                   jax.ShapeDtypeStruct((NP, 1), F32),
                   jax.ShapeDtypeStruct((NP, 1), F32),
                   jax.ShapeDtypeStruct((NP, 1), F32),
                   jax.ShapeDtypeStruct((NP, 2), F32),
                   jax.ShapeDtypeStruct((1, NP), F32),
                   jax.ShapeDtypeStruct((G, 1), F32),
                   jax.ShapeDtypeStruct((G, H), F32),
                   jax.ShapeDtypeStruct((G, 1), F32)],
    )(hT, aggT, lnG, lnB, batch_row, *hw)


def _k5_body(att_ref, batch_ref, m_ref, ex_ref, den_ref):
    i = pl.program_id(0)
    att = att_ref[...]
    bb = batch_ref[...]
    gio = lax.broadcasted_iota(jnp.int32, (G, NB), 0).astype(F32)
    mask = (gio == bb).astype(F32)
    mcol = _dgch(m_ref[...], mask, 0, 0)
    valid = bb < float(G)
    ex = jnp.where(valid, jnp.exp(att - mcol), 0.0)
    ex_ref[...] = ex
    bden = _dgch(mask, ex, 1, 1)

    @pl.when(i == 0)
    def _():
        den_ref[...] = bden

    @pl.when(i > 0)
    def _():
        den_ref[...] += bden


def _k5(att_row, batch_row, m):
    full = lambda shp: pl.BlockSpec(shp, lambda i: tuple(0 for _ in shp))
    blkR = pl.BlockSpec((1, NB), lambda i: (0, i))
    return pl.pallas_call(
        _k5_body,
        grid=(NBLK,),
        in_specs=[blkR, blkR, full((G, 1))],
        out_specs=[blkR, full((G, 1))],
        out_shape=[jax.ShapeDtypeStruct((1, NP), F32),
                   jax.ShapeDtypeStruct((G, 1), F32)],
    )(att_row, batch_row, m)


def _k6_body(ex_ref, batch_ref, den_ref, h_ref, ssum_ref, cnt_ref,
             costW1, costB1, costW2, costB2, clsW1, clsB1, clsW2, clsB2,
             attw_ref, gsum_ref, cost_ref, logits_ref):
    i = pl.program_id(0)
    bb = batch_ref[...]
    gio = lax.broadcasted_iota(jnp.int32, (G, NB), 0).astype(F32)
    mask = (gio == bb).astype(F32)
    denc = _dgch(den_ref[...], mask, 0, 0)
    valid = bb < float(G)
    attw = jnp.where(valid, ex_ref[...] / jnp.maximum(denc, 1e-30), 0.0)
    attw_ref[...] = attw
    bg = _dgch(mask * attw, h_ref[...], 1, 0)

    @pl.when(i == 0)
    def _():
        gsum_ref[...] = bg

    @pl.when(i > 0)
    def _():
        gsum_ref[...] += bg

    gs = gsum_ref[...]
    gmean = ssum_ref[...] / jnp.maximum(cnt_ref[...], 1.0)
    cost_ref[...] = _dg(jax.nn.relu(
        _dg(gmean, costW1[...], (((1,), (0,)), ((), ()))) + costB1[...]),
        costW2[...], (((1,), (0,)), ((), ()))) + costB2[...]
    logits_ref[...] = _dg(jax.nn.relu(
        _dg(gs, clsW1[...], (((1,), (0,)), ((), ()))) + clsB1[...]),
        clsW2[...], (((1,), (0,)), ((), ()))) + clsB2[...]


def _k6(ex_row, batch_row, den, h, ssum, cnt, hw):
    full = lambda shp: pl.BlockSpec(shp, lambda i: tuple(0 for _ in shp))
    blkR = pl.BlockSpec((1, NB), lambda i: (0, i))
    return pl.pallas_call(
        _k6_body,
        grid=(NBLK,),
        in_specs=[blkR, blkR, full((G, 1)),
                  pl.BlockSpec((NB, H), lambda i: (i, 0)),
                  full((G, H)), full((G, 1))] + [full(a.shape) for a in hw],
        out_specs=[blkR, full((G, H)), full((G, 1)), full((G, 3))],
        out_shape=[jax.ShapeDtypeStruct((1, NP), F32),
                   jax.ShapeDtypeStruct((G, H), F32),
                   jax.ShapeDtypeStruct((G, 1), F32),
                   jax.ShapeDtypeStruct((G, 3), F32)],
    )(ex_row, batch_row, den, h, ssum, cnt, *hw)


def _edge_stage(a_flat, pk, et_flat):
    return _sc_edge_kernel()(a_flat, pk, et_flat)


def kernel(x, edge_index, edge_attr, batch, params):
    p = params
    x_pad = jnp.pad(x, ((0, NP - N), (0, 0)))
    batch_row = jnp.pad(batch, (0, NP - N), constant_values=G) \
                   .astype(F32).reshape(1, NP)
    src2 = jnp.pad(edge_index[0].astype(jnp.int32), (0, EP - E)) \
              .reshape(157, 2048)
    dst2 = jnp.pad(edge_index[1].astype(jnp.int32), (0, EP - E),
                   constant_values=NP - 1).reshape(157, 2048)
    pk = _kpk(src2, dst2).reshape(-1)

    lps = p['layers']
    ea_pad = jnp.pad(edge_attr, ((0, EP - E), (0, 0)))
    ewS = jnp.stack([lp['edge_W'] for lp in lps])
    ebS = jnp.stack([lp['edge_b'].reshape(1, H) for lp in lps])
    weS = jnp.stack([lp['msg_W'][H:] for lp in lps])
    mbS = jnp.stack([lp['msg_b'].reshape(H, 1) for lp in lps])
    etT = _ket(ea_pad, ewS, ebS, weS, mbS)

    hT, aT = _k1(x_pad, p['proj_W'], p['proj_b'].reshape(H, 1),
                 lps[0]['node_W'], lps[0]['node_b'].reshape(H, 1),
                 lps[0]['msg_W'][:H])

    for l in range(L):
        aggT = _edge_stage(aT.reshape(-1), pk,
                           etT[l].reshape(-1)).reshape(H, NP)
        if l < L - 1:
            lp = lps[l + 1]
            hT, aT = _k2(hT, aggT, lps[l]['ln_g'].reshape(H, 1),
                         lps[l]['ln_b'].reshape(H, 1),
                         lp['node_W'], lp['node_b'].reshape(H, 1),
                         lp['msg_W'][:H])

    hw3 = [p['pf_W1'], p['pf_b1'].reshape(1, H), p['pf_W2'],
           p['pf_b2'].reshape(1, 3),
           p['gen_W1'], p['gen_b1'].reshape(1, H), p['gen_W2'],
           p['gen_b2'].reshape(1, 2),
           p['att_W1'], p['att_b1'].reshape(H // 2, 1), p['att_W2'],
           p['att_b2'].reshape(1, 1)]
    (h, vmag, s, c, gen, att_row, m, ssum, cnt) = _k3(
        hT, aggT, lps[2]['ln_g'].reshape(H, 1), lps[2]['ln_b'].reshape(H, 1),
        batch_row, hw3)

    ex_row, den = _k5(att_row, batch_row, m)

    hw6 = [p['cost_W1'], p['cost_b1'].reshape(1, H // 2), p['cost_W2'],
           p['cost_b2'].reshape(1, 1),
           p['cls_W1'], p['cls_b1'].reshape(1, H // 2), p['cls_W2'],
           p['cls_b2'].reshape(1, 3)]
    attw_row, _gsum, cost, logits = _k6(ex_row, batch_row, den, h, ssum,
                                        cnt, hw6)

    return (vmag[:N], s[:N], c[:N], gen[:N], cost, logits,
            attw_row.reshape(NP, 1)[:N], h[:N])
